# baseline (speedup 1.0000x reference)
import jax
import jax.numpy as jnp
from jax import lax
from jax.experimental import pallas as pl
from jax.experimental.pallas import tpu as pltpu
from jax.experimental.pallas import tpu_sc as plsc

N = 10000
E = 320000
F_IN = 256
D = 128

C = 100
NW = 32
NCPW = E // (NW * C)
NPAD = 10240
RB = 1024
PC = 80

f32 = jnp.float32
i32 = jnp.int32

_mesh = plsc.VectorSubcoreMesh(core_axis_name="c", subcore_axis_name="s")


def _deg_body(e0, e1, e2, d0, d1, d2,
              srcb, dstb, ones, zst,
              a00, a01, a10, a11, a20, a21):
    cc = lax.axis_index("c")
    s = lax.axis_index("s")
    wid = s * 2 + cc
    accs = (a00, a01, a10, a11, a20, a21)

    @pl.loop(0, 64)
    def _(i):
        zst[pl.ds(i * 16, 16)] = jnp.zeros((16,), f32)

    @pl.loop(0, 7)
    def _(i):
        ones[pl.ds(i * 16, 16)] = jnp.ones((16,), f32)

    @pl.when(s < 10)
    def _():
        for acc in accs:
            pltpu.sync_copy(zst, acc.at[pl.ds(s * 1024, 1024)])
    plsc.subcore_barrier()

    for e, asrc, adst in ((e0, a00, a01), (e1, a10, a11), (e2, a20, a21)):
        for q in range(5):
            pltpu.sync_copy(e.at[0, wid, q], srcb)
            pltpu.sync_copy(e.at[1, wid, q], dstb)

            @pl.loop(0, NCPW // 5)
            def _(j, _as=asrc, _ad=adst):
                pltpu.sync_copy(ones.at[pl.ds(0, C)], _as.at[srcb.at[j]],
                                add=True)
                pltpu.sync_copy(ones.at[pl.ds(0, C)], _ad.at[dstb.at[j]],
                                add=True)

    plsc.subcore_barrier()

    @pl.when(s < 10)
    def _():
        for d, pair in ((d0, (a00, a01)), (d1, (a10, a11)), (d2, (a20, a21))):
            for role in range(2):
                pltpu.sync_copy(pair[role].at[pl.ds(s * 1024, 1024)], zst)
                pltpu.sync_copy(zst,
                                d.at[cc, role, 0, pl.ds(s * 1024, 1024)])


_deg_call = pl.kernel(
    _deg_body,
    out_type=tuple(jax.ShapeDtypeStruct((2, 2, 1, NPAD), f32)
                   for _ in range(3)),
    mesh=_mesh,
    scratch_types=[
        pltpu.VMEM((NCPW // 5, C), i32),
        pltpu.VMEM((NCPW // 5, C), i32),
        pltpu.VMEM((112,), f32),
        pltpu.VMEM((1024,), f32),
    ] + [pltpu.VMEM_SHARED((NPAD,), f32) for _ in range(6)],
)


def _mult_rows(rows, ewb, j):
    @plsc.parallel_loop(0, 6)
    def _(g):
        wv = ewb[j, pl.ds(g * 16, 16)]
        for ii in range(16):
            w = wv[ii]
            r = g * 16 + ii
            for k in range(8):
                sl = pl.ds(k * 16, 16)
                rows[r, sl] = rows[r, sl] * w

    wv = ewb[j, pl.ds(84, 16)]
    for ii in range(4):
        w = wv[12 + ii]
        r = 96 + ii
        for k in range(8):
            sl = pl.ds(k * 16, 16)
            rows[r, sl] = rows[r, sl] * w


NB = 5
BCH = NCPW // NB


def _conv_body(table, eidx, ew, out, srcb, dstb, ewb, rows0, rows1, acc,
               sg0, sg1, ss0, ss1):
    cc = lax.axis_index("c")
    s = lax.axis_index("s")
    wid = s * 2 + cc

    @pl.loop(0, 64)
    def _(i):
        for k in range(8):
            rows0[i, pl.ds(k * 16, 16)] = jnp.zeros((16,), f32)

    for k in range(10):
        pltpu.sync_copy(rows0.at[pl.ds(0, 64), :],
                        acc.at[pl.ds(s * 640 + k * 64, 64), :])
    plsc.subcore_barrier()

    for q in range(NB):
        pltpu.sync_copy(eidx.at[0, wid, q], srcb)
        pltpu.sync_copy(eidx.at[1, wid, q], dstb)
        pltpu.sync_copy(ew.at[wid, q], ewb)
        pltpu.async_copy(table.at[srcb.at[0]], rows0, sg0)
        pltpu.async_copy(table.at[srcb.at[1]], rows1, sg1)

        @pl.loop(0, BCH // 2)
        def _(jj):
            j0 = 2 * jj
            j1 = j0 + 1
            pltpu.make_async_copy(table.at[srcb.at[j0]], rows0, sg0).wait()
            _mult_rows(rows0, ewb, j0)
            pltpu.async_copy(rows0, acc.at[dstb.at[j0]], ss0, add=True)
            pltpu.make_async_copy(table.at[srcb.at[j1]], rows1, sg1).wait()

            @pl.when(jj < BCH // 2 - 1)
            def _():
                pltpu.make_async_copy(rows0, acc.at[dstb.at[j0]], ss0).wait()
                pltpu.async_copy(table.at[srcb.at[j0 + 2]], rows0, sg0)

            _mult_rows(rows1, ewb, j1)
            pltpu.async_copy(rows1, acc.at[dstb.at[j1]], ss1, add=True)

            @pl.when(jj < BCH // 2 - 1)
            def _():
                pltpu.make_async_copy(rows1, acc.at[dstb.at[j1]], ss1).wait()
                pltpu.async_copy(table.at[srcb.at[j1 + 2]], rows1, sg1)

        pltpu.make_async_copy(rows0, acc.at[dstb.at[BCH - 2]], ss0).wait()
        pltpu.make_async_copy(rows1, acc.at[dstb.at[BCH - 1]], ss1).wait()

    plsc.subcore_barrier()
    for k in range(10):
        off = s * 640 + k * 64
        pltpu.sync_copy(acc.at[pl.ds(off, 64), :], rows0.at[pl.ds(0, 64), :])
        pltpu.sync_copy(rows0.at[pl.ds(0, 64), :],
                        out.at[cc, pl.ds(off, 64), :])


_conv_call = pl.kernel(
    _conv_body,
    out_type=jax.ShapeDtypeStruct((2, NPAD, D), f32),
    mesh=_mesh,
    scratch_types=[
        pltpu.VMEM((BCH, C), i32),
        pltpu.VMEM((BCH, C), i32),
        pltpu.VMEM((BCH, C), f32),
        pltpu.VMEM((C, D), f32),
        pltpu.VMEM((C, D), f32),
        pltpu.VMEM_SHARED((NPAD, D), f32),
        pltpu.SemaphoreType.DMA,
        pltpu.SemaphoreType.DMA,
        pltpu.SemaphoreType.DMA,
        pltpu.SemaphoreType.DMA,
    ],
)


def _pool_body(h0, h1, h2, idx3, idxp, out, rawb, clampb, flags, recipb,
               r0, r1, r2, outb):
    cc = lax.axis_index("c")
    s = lax.axis_index("s")
    wid = s * 2 + cc
    hs = (h0, h1, h2)
    rs = (r0, r1, r2)
    for k in range(4):
        q = wid * 4 + k
        rbase = wid * 320 + k * PC
        for t in range(3):
            pltpu.sync_copy(idx3.at[t, q], rawb.at[t])
            pltpu.sync_copy(idxp.at[t, q], clampb.at[t])
        for v in range(5):
            sl = pl.ds(v * 16, 16)
            cnt = jnp.zeros((16,), f32)
            for t in range(3):
                r = rawb[t, 0, sl]
                fl = jnp.where(r >= 0, 1.0, 0.0).astype(f32)
                flags[t, 0, sl] = fl
                cnt = cnt + fl
            recipb[sl] = 1.0 / jnp.maximum(cnt, 1.0)
        for t in range(3):
            pltpu.sync_copy(hs[t].at[clampb.at[t, 0]], rs[t])

        @pl.loop(0, 5)
        def _(g):
            w0v = flags[0, 0, pl.ds(g * 16, 16)]
            w1v = flags[1, 0, pl.ds(g * 16, 16)]
            w2v = flags[2, 0, pl.ds(g * 16, 16)]
            rrv = recipb[pl.ds(g * 16, 16)]
            for ii in range(16):
                r = g * 16 + ii
                w0 = w0v[ii]
                w1 = w1v[ii]
                w2 = w2v[ii]
                rr = rrv[ii]
                for kk in range(8):
                    sl = pl.ds(kk * 16, 16)
                    outb[r, sl] = (r0[r, sl] * w0 + r1[r, sl] * w1
                                   + r2[r, sl] * w2) * rr

        pltpu.sync_copy(outb, out.at[pl.ds(rbase, PC), :])


_pool_call = pl.kernel(
    _pool_body,
    out_type=jax.ShapeDtypeStruct((NPAD, D), f32),
    mesh=_mesh,
    scratch_types=[
        pltpu.VMEM((3, 1, PC), i32),
        pltpu.VMEM((3, 1, PC), i32),
        pltpu.VMEM((3, 1, PC), f32),
        pltpu.VMEM((PC,), f32),
        pltpu.VMEM((PC, D), f32),
        pltpu.VMEM((PC, D), f32),
        pltpu.VMEM((PC, D), f32),
        pltpu.VMEM((PC, D), f32),
    ],
)


def _rs_body(d0, d1, d2, r0, r1, r2):
    for dref, rref in ((d0, r0), (d1, r1), (d2, r2)):
        p = dref[...]
        deg = p[0, :, 0, :] + p[1, :, 0, :]
        rref[...] = lax.rsqrt(jnp.maximum(deg, 1.0))


def _rs_call(d0, d1, d2):
    return pl.pallas_call(
        _rs_body,
        out_shape=tuple(jax.ShapeDtypeStruct((2, NPAD), f32)
                        for _ in range(3)),
    )(d0, d1, d2)


def _full(shape):
    return pl.BlockSpec(shape, lambda i: tuple(0 for _ in shape))


def _ka_body(feat, rs, win, bin_, c1w, out):
    h = jnp.maximum(
        jnp.dot(feat[...], win[...], preferred_element_type=f32) + bin_[...],
        0.0)
    xw = jnp.dot(h, c1w[...], preferred_element_type=f32)
    out[...] = xw * rs[0, :][:, None]


def _ka(feat, rs, win, bin_, c1w):
    return pl.pallas_call(
        _ka_body,
        grid=(pl.cdiv(N, RB),),
        in_specs=[
            pl.BlockSpec((RB, F_IN), lambda i: (i, 0)),
            pl.BlockSpec((2, RB), lambda i: (0, i)),
            _full((F_IN, D)),
            _full((D,)),
            _full((D, D)),
        ],
        out_specs=pl.BlockSpec((RB, D), lambda i: (i, 0)),
        out_shape=jax.ShapeDtypeStruct((N, D), f32),
    )(feat, rs, win, bin_, c1w)


def _kb_body(aggp, rs, c1b, c2w, out):
    p = aggp[...]
    h1 = jnp.maximum(
        (p[0] + p[1]) * rs[1, :][:, None] + c1b[...], 0.0)
    out[...] = jnp.dot(h1, c2w[...],
                       preferred_element_type=f32) * rs[0, :][:, None]


def _kb(aggp, rs, c1b, c2w):
    return pl.pallas_call(
        _kb_body,
        grid=(pl.cdiv(N, RB),),
        in_specs=[
            pl.BlockSpec((2, RB, D), lambda i: (0, i, 0)),
            pl.BlockSpec((2, RB), lambda i: (0, i)),
            _full((D,)),
            _full((D, D)),
        ],
        out_specs=pl.BlockSpec((RB, D), lambda i: (i, 0)),
        out_shape=jax.ShapeDtypeStruct((N, D), f32),
    )(aggp, rs, c1b, c2w)


def _kc_body(aggp, rs, c2b, out):
    p = aggp[...]
    out[...] = (p[0] + p[1]) * rs[1, :][:, None] + c2b[...]


def _kc(aggp, rs, c2b):
    return pl.pallas_call(
        _kc_body,
        grid=(pl.cdiv(N, RB),),
        in_specs=[
            pl.BlockSpec((2, RB, D), lambda i: (0, i, 0)),
            pl.BlockSpec((2, RB), lambda i: (0, i)),
            _full((D,)),
        ],
        out_specs=pl.BlockSpec((RB, D), lambda i: (i, 0)),
        out_shape=jax.ShapeDtypeStruct((N, D), f32),
    )(aggp, rs, c2b)


def _mlp_body(x, w1, b1, w2, b2, w3, b3, out):
    z = jnp.maximum(jnp.dot(x[...], w1[...],
                            preferred_element_type=f32) + b1[...], 0.0)
    z = jnp.maximum(jnp.dot(z, w2[...],
                            preferred_element_type=f32) + b2[...], 0.0)
    out[...] = jnp.dot(z, w3[...], preferred_element_type=f32) + b3[...]


def _mlp(x, w1, b1, w2, b2, w3, b3):
    return pl.pallas_call(
        _mlp_body,
        grid=(NPAD // RB,),
        in_specs=[pl.BlockSpec((RB, D), lambda i: (i, 0))]
        + [_full((D, D)), _full((D,))] * 3,
        out_specs=pl.BlockSpec((RB, D), lambda i: (i, 0)),
        out_shape=jax.ShapeDtypeStruct((NPAD, D), f32),
    )(x, w1, b1, w2, b2, w3, b3)


def kernel(feat_t0, feat_t1, feat_t2,
           edge_index_r0, edge_index_r1, edge_index_r2,
           ew_r0, ew_r1, ew_r2,
           index_matrix, W_in, b_in,
           c1W_r0, c1W_r1, c1W_r2, c1b_r0, c1b_r1, c1b_r2,
           c2W_r0, c2W_r1, c2W_r2, c2b_r0, c2b_r1, c2b_r2,
           mW1, mb1, mW2, mb2, mW3, mb3):
    feats = (feat_t0, feat_t1, feat_t2)
    eidx = [e.reshape(2, NW, NB, BCH, C)
            for e in (edge_index_r0, edge_index_r1, edge_index_r2)]
    ews = [w.reshape(NW, NB, BCH, C) for w in (ew_r0, ew_r1, ew_r2)]
    c1W = (c1W_r0, c1W_r1, c1W_r2)
    c1b = (c1b_r0, c1b_r1, c1b_r2)
    c2W = (c2W_r0, c2W_r1, c2W_r2)
    c2b = (c2b_r0, c2b_r1, c2b_r2)

    degp = _deg_call(eidx[0], eidx[1], eidx[2])
    rs = _rs_call(*degp)

    h2 = []
    for t in range(3):
        xw1 = _ka(feats[t], rs[t], W_in, b_in, c1W[t])
        aggp1 = _conv_call(xw1, eidx[t], ews[t])
        xw2 = _kb(aggp1, rs[t], c1b[t], c2W[t])
        aggp2 = _conv_call(xw2, eidx[t], ews[t])
        h2.append(_kc(aggp2, rs[t], c2b[t]))

    imt = jnp.pad(index_matrix.T, ((0, 0), (0, NPAD - N)),
                  constant_values=-1)
    spread = jnp.arange(NPAD, dtype=i32)[None, :] & 8191
    idx3 = imt.reshape(3, 128, 1, PC)
    idxp = jnp.where(imt >= 0, imt, spread).reshape(3, 128, 1, PC)
    mean = _pool_call(h2[0], h2[1], h2[2], idx3, idxp)
    out = _mlp(mean, mW1, mb1, mW2, mb2, mW3, mb3)
    return out[:N]


# speedup vs baseline: 1.0075x; 1.0075x over previous
import jax
import jax.numpy as jnp
from jax import lax
from jax.experimental import pallas as pl
from jax.experimental.pallas import tpu as pltpu
from jax.experimental.pallas import tpu_sc as plsc

N = 10000
E = 320000
F_IN = 256
D = 128

C = 100
NW = 32
NCPW = E // (NW * C)
NPAD = 10240
RB = 1024
PC = 80

f32 = jnp.float32
i32 = jnp.int32

_mesh = plsc.VectorSubcoreMesh(core_axis_name="c", subcore_axis_name="s")


def _deg_body(e0, e1, e2, d0, d1, d2,
              srcb, dstb, ones, zst,
              a00, a01, a10, a11, a20, a21):
    cc = lax.axis_index("c")
    s = lax.axis_index("s")
    wid = s * 2 + cc
    accs = (a00, a01, a10, a11, a20, a21)

    @pl.loop(0, 64)
    def _(i):
        zst[pl.ds(i * 16, 16)] = jnp.zeros((16,), f32)

    @pl.loop(0, 7)
    def _(i):
        ones[pl.ds(i * 16, 16)] = jnp.ones((16,), f32)

    @pl.when(s < 10)
    def _():
        for acc in accs:
            pltpu.sync_copy(zst, acc.at[pl.ds(s * 1024, 1024)])
    plsc.subcore_barrier()

    for e, asrc, adst in ((e0, a00, a01), (e1, a10, a11), (e2, a20, a21)):
        for q in range(5):
            pltpu.sync_copy(e.at[0, wid, q], srcb)
            pltpu.sync_copy(e.at[1, wid, q], dstb)

            @pl.loop(0, NCPW // 5)
            def _(j, _as=asrc, _ad=adst):
                pltpu.sync_copy(ones.at[pl.ds(0, C)], _as.at[srcb.at[j]],
                                add=True)
                pltpu.sync_copy(ones.at[pl.ds(0, C)], _ad.at[dstb.at[j]],
                                add=True)

    plsc.subcore_barrier()

    @pl.when(s < 10)
    def _():
        for d, pair in ((d0, (a00, a01)), (d1, (a10, a11)), (d2, (a20, a21))):
            for role in range(2):
                pltpu.sync_copy(pair[role].at[pl.ds(s * 1024, 1024)], zst)
                pltpu.sync_copy(zst,
                                d.at[cc, role, 0, pl.ds(s * 1024, 1024)])


_deg_call = pl.kernel(
    _deg_body,
    out_type=tuple(jax.ShapeDtypeStruct((2, 2, 1, NPAD), f32)
                   for _ in range(3)),
    mesh=_mesh,
    scratch_types=[
        pltpu.VMEM((NCPW // 5, C), i32),
        pltpu.VMEM((NCPW // 5, C), i32),
        pltpu.VMEM((112,), f32),
        pltpu.VMEM((1024,), f32),
    ] + [pltpu.VMEM_SHARED((NPAD,), f32) for _ in range(6)],
)


def _mult_rows(rows, ewb, j):
    @plsc.parallel_loop(0, 6)
    def _(g):
        wv = ewb[j, pl.ds(g * 16, 16)]
        for ii in range(16):
            w = wv[ii]
            r = g * 16 + ii
            for k in range(8):
                sl = pl.ds(k * 16, 16)
                rows[r, sl] = rows[r, sl] * w

    wv = ewb[j, pl.ds(84, 16)]
    for ii in range(4):
        w = wv[12 + ii]
        r = 96 + ii
        for k in range(8):
            sl = pl.ds(k * 16, 16)
            rows[r, sl] = rows[r, sl] * w


NB = 5
BCH = NCPW // NB


def _conv_body(table, eidx, ew, out, srcb, dstb, ewb, rows0, rows1, acc,
               sg0, sg1, ss0, ss1):
    cc = lax.axis_index("c")
    s = lax.axis_index("s")
    wid = s * 2 + cc

    @pl.loop(0, 64)
    def _(i):
        for k in range(8):
            rows0[i, pl.ds(k * 16, 16)] = jnp.zeros((16,), f32)

    for k in range(10):
        pltpu.sync_copy(rows0.at[pl.ds(0, 64), :],
                        acc.at[pl.ds(s * 640 + k * 64, 64), :])
    plsc.subcore_barrier()

    for q in range(NB):
        pltpu.sync_copy(eidx.at[0, wid, q], srcb)
        pltpu.sync_copy(eidx.at[1, wid, q], dstb)
        pltpu.sync_copy(ew.at[wid, q], ewb)
        pltpu.async_copy(table.at[srcb.at[0]], rows0, sg0)
        pltpu.async_copy(table.at[srcb.at[1]], rows1, sg1)

        @pl.loop(0, BCH // 2)
        def _(jj):
            j0 = 2 * jj
            j1 = j0 + 1
            pltpu.make_async_copy(table.at[srcb.at[j0]], rows0, sg0).wait()
            _mult_rows(rows0, ewb, j0)
            pltpu.async_copy(rows0, acc.at[dstb.at[j0]], ss0, add=True)
            pltpu.make_async_copy(table.at[srcb.at[j1]], rows1, sg1).wait()

            @pl.when(jj < BCH // 2 - 1)
            def _():
                pltpu.make_async_copy(rows0, acc.at[dstb.at[j0]], ss0).wait()
                pltpu.async_copy(table.at[srcb.at[j0 + 2]], rows0, sg0)

            _mult_rows(rows1, ewb, j1)
            pltpu.async_copy(rows1, acc.at[dstb.at[j1]], ss1, add=True)

            @pl.when(jj < BCH // 2 - 1)
            def _():
                pltpu.make_async_copy(rows1, acc.at[dstb.at[j1]], ss1).wait()
                pltpu.async_copy(table.at[srcb.at[j1 + 2]], rows1, sg1)

        pltpu.make_async_copy(rows0, acc.at[dstb.at[BCH - 2]], ss0).wait()
        pltpu.make_async_copy(rows1, acc.at[dstb.at[BCH - 1]], ss1).wait()

    plsc.subcore_barrier()
    for k in range(10):
        off = s * 640 + k * 64
        pltpu.sync_copy(acc.at[pl.ds(off, 64), :], rows0.at[pl.ds(0, 64), :])
        pltpu.sync_copy(rows0.at[pl.ds(0, 64), :],
                        out.at[cc, pl.ds(off, 64), :])


_conv_call = pl.kernel(
    _conv_body,
    out_type=jax.ShapeDtypeStruct((2, NPAD, D), f32),
    mesh=_mesh,
    scratch_types=[
        pltpu.VMEM((BCH, C), i32),
        pltpu.VMEM((BCH, C), i32),
        pltpu.VMEM((BCH, C), f32),
        pltpu.VMEM((C, D), f32),
        pltpu.VMEM((C, D), f32),
        pltpu.VMEM_SHARED((NPAD, D), f32),
        pltpu.SemaphoreType.DMA,
        pltpu.SemaphoreType.DMA,
        pltpu.SemaphoreType.DMA,
        pltpu.SemaphoreType.DMA,
    ],
)


def _pool_body(h0, h1, h2, idx3, out, rawb, clampb, flags, recipb,
               r0, r1, r2, outb):
    cc = lax.axis_index("c")
    s = lax.axis_index("s")
    wid = s * 2 + cc
    hs = (h0, h1, h2)
    rs = (r0, r1, r2)
    for k in range(4):
        q = wid * 4 + k
        rbase = wid * 320 + k * PC
        for t in range(3):
            pltpu.sync_copy(idx3.at[t, q], rawb.at[t])
        for v in range(5):
            sl = pl.ds(v * 16, 16)
            cnt = jnp.zeros((16,), f32)
            spread = (lax.iota(i32, 16) + rbase + v * 16) & 8191
            for t in range(3):
                r = rawb[t, 0, sl]
                clampb[t, 0, sl] = jnp.where(r >= 0, r, spread)
                fl = jnp.where(r >= 0, 1.0, 0.0).astype(f32)
                flags[t, 0, sl] = fl
                cnt = cnt + fl
            recipb[sl] = 1.0 / jnp.maximum(cnt, 1.0)
        for t in range(3):
            pltpu.sync_copy(hs[t].at[clampb.at[t, 0]], rs[t])

        @pl.loop(0, 5)
        def _(g):
            w0v = flags[0, 0, pl.ds(g * 16, 16)]
            w1v = flags[1, 0, pl.ds(g * 16, 16)]
            w2v = flags[2, 0, pl.ds(g * 16, 16)]
            rrv = recipb[pl.ds(g * 16, 16)]
            for ii in range(16):
                r = g * 16 + ii
                w0 = w0v[ii]
                w1 = w1v[ii]
                w2 = w2v[ii]
                rr = rrv[ii]
                for kk in range(8):
                    sl = pl.ds(kk * 16, 16)
                    outb[r, sl] = (r0[r, sl] * w0 + r1[r, sl] * w1
                                   + r2[r, sl] * w2) * rr

        pltpu.sync_copy(outb, out.at[pl.ds(rbase, PC), :])


_pool_call = pl.kernel(
    _pool_body,
    out_type=jax.ShapeDtypeStruct((NPAD, D), f32),
    mesh=_mesh,
    scratch_types=[
        pltpu.VMEM((3, 1, PC), i32),
        pltpu.VMEM((3, 1, PC), i32),
        pltpu.VMEM((3, 1, PC), f32),
        pltpu.VMEM((PC,), f32),
        pltpu.VMEM((PC, D), f32),
        pltpu.VMEM((PC, D), f32),
        pltpu.VMEM((PC, D), f32),
        pltpu.VMEM((PC, D), f32),
    ],
)


def _rs_body(d0, d1, d2, r0, r1, r2):
    for dref, rref in ((d0, r0), (d1, r1), (d2, r2)):
        p = dref[...]
        deg = p[0, :, 0, :] + p[1, :, 0, :]
        rref[...] = lax.rsqrt(jnp.maximum(deg, 1.0))


def _rs_call(d0, d1, d2):
    return pl.pallas_call(
        _rs_body,
        out_shape=tuple(jax.ShapeDtypeStruct((2, NPAD), f32)
                        for _ in range(3)),
    )(d0, d1, d2)


def _full(shape):
    return pl.BlockSpec(shape, lambda i: tuple(0 for _ in shape))


def _ka_body(feat, rs, win, bin_, c1w, out):
    h = jnp.maximum(
        jnp.dot(feat[...], win[...], preferred_element_type=f32) + bin_[...],
        0.0)
    xw = jnp.dot(h, c1w[...], preferred_element_type=f32)
    out[...] = xw * rs[0, :][:, None]


def _ka(feat, rs, win, bin_, c1w):
    return pl.pallas_call(
        _ka_body,
        grid=(pl.cdiv(N, RB),),
        in_specs=[
            pl.BlockSpec((RB, F_IN), lambda i: (i, 0)),
            pl.BlockSpec((2, RB), lambda i: (0, i)),
            _full((F_IN, D)),
            _full((D,)),
            _full((D, D)),
        ],
        out_specs=pl.BlockSpec((RB, D), lambda i: (i, 0)),
        out_shape=jax.ShapeDtypeStruct((N, D), f32),
    )(feat, rs, win, bin_, c1w)


def _kb_body(aggp, rs, c1b, c2w, out):
    p = aggp[...]
    h1 = jnp.maximum(
        (p[0] + p[1]) * rs[1, :][:, None] + c1b[...], 0.0)
    out[...] = jnp.dot(h1, c2w[...],
                       preferred_element_type=f32) * rs[0, :][:, None]


def _kb(aggp, rs, c1b, c2w):
    return pl.pallas_call(
        _kb_body,
        grid=(pl.cdiv(N, RB),),
        in_specs=[
            pl.BlockSpec((2, RB, D), lambda i: (0, i, 0)),
            pl.BlockSpec((2, RB), lambda i: (0, i)),
            _full((D,)),
            _full((D, D)),
        ],
        out_specs=pl.BlockSpec((RB, D), lambda i: (i, 0)),
        out_shape=jax.ShapeDtypeStruct((N, D), f32),
    )(aggp, rs, c1b, c2w)


def _kc_body(aggp, rs, c2b, out):
    p = aggp[...]
    out[...] = (p[0] + p[1]) * rs[1, :][:, None] + c2b[...]


def _kc(aggp, rs, c2b):
    return pl.pallas_call(
        _kc_body,
        grid=(pl.cdiv(N, RB),),
        in_specs=[
            pl.BlockSpec((2, RB, D), lambda i: (0, i, 0)),
            pl.BlockSpec((2, RB), lambda i: (0, i)),
            _full((D,)),
        ],
        out_specs=pl.BlockSpec((RB, D), lambda i: (i, 0)),
        out_shape=jax.ShapeDtypeStruct((N, D), f32),
    )(aggp, rs, c2b)


def _mlp_body(x, w1, b1, w2, b2, w3, b3, out):
    z = jnp.maximum(jnp.dot(x[...], w1[...],
                            preferred_element_type=f32) + b1[...], 0.0)
    z = jnp.maximum(jnp.dot(z, w2[...],
                            preferred_element_type=f32) + b2[...], 0.0)
    out[...] = jnp.dot(z, w3[...], preferred_element_type=f32) + b3[...]


def _mlp(x, w1, b1, w2, b2, w3, b3):
    return pl.pallas_call(
        _mlp_body,
        grid=(NPAD // RB,),
        in_specs=[pl.BlockSpec((RB, D), lambda i: (i, 0))]
        + [_full((D, D)), _full((D,))] * 3,
        out_specs=pl.BlockSpec((RB, D), lambda i: (i, 0)),
        out_shape=jax.ShapeDtypeStruct((NPAD, D), f32),
    )(x, w1, b1, w2, b2, w3, b3)


def kernel(feat_t0, feat_t1, feat_t2,
           edge_index_r0, edge_index_r1, edge_index_r2,
           ew_r0, ew_r1, ew_r2,
           index_matrix, W_in, b_in,
           c1W_r0, c1W_r1, c1W_r2, c1b_r0, c1b_r1, c1b_r2,
           c2W_r0, c2W_r1, c2W_r2, c2b_r0, c2b_r1, c2b_r2,
           mW1, mb1, mW2, mb2, mW3, mb3):
    feats = (feat_t0, feat_t1, feat_t2)
    eidx = [e.reshape(2, NW, NB, BCH, C)
            for e in (edge_index_r0, edge_index_r1, edge_index_r2)]
    ews = [w.reshape(NW, NB, BCH, C) for w in (ew_r0, ew_r1, ew_r2)]
    c1W = (c1W_r0, c1W_r1, c1W_r2)
    c1b = (c1b_r0, c1b_r1, c1b_r2)
    c2W = (c2W_r0, c2W_r1, c2W_r2)
    c2b = (c2b_r0, c2b_r1, c2b_r2)

    degp = _deg_call(eidx[0], eidx[1], eidx[2])
    rs = _rs_call(*degp)

    h2 = []
    for t in range(3):
        xw1 = _ka(feats[t], rs[t], W_in, b_in, c1W[t])
        aggp1 = _conv_call(xw1, eidx[t], ews[t])
        xw2 = _kb(aggp1, rs[t], c1b[t], c2W[t])
        aggp2 = _conv_call(xw2, eidx[t], ews[t])
        h2.append(_kc(aggp2, rs[t], c2b[t]))

    idx3 = jnp.pad(index_matrix.T, ((0, 0), (0, NPAD - N)),
                   constant_values=-1).reshape(3, 128, 1, PC)
    mean = _pool_call(h2[0], h2[1], h2[2], idx3)
    out = _mlp(mean, mW1, mb1, mW2, mb2, mW3, mb3)
    return out[:N]
